# baseline (speedup 1.0000x reference)
import functools

import jax
import jax.numpy as jnp
from jax.experimental import pallas as pl
from jax.experimental.pallas import tpu as pltpu

_HB = 8
_NBUF = 12


def _ecc_manual_kernel(x_hbm, proto_ref, out_ref, xbuf, sems, *,
                       num_classes, hb, nbuf):
    i = pl.program_id(0)
    n = pl.num_programs(0)
    nh = out_ref.shape[2] // hb

    def issue(j):
        jb = j // nh
        jh = j % nh
        slot = j % nbuf
        pltpu.make_async_copy(
            x_hbm.at[jb, :, pl.ds(jh * hb, hb), :],
            xbuf.at[slot],
            sems.at[slot],
        ).start()

    @pl.when(i == 0)
    def _():
        for j in range(nbuf):
            issue(j)

    @pl.when((i > 0) & (i + nbuf - 1 < n))
    def _():
        issue(i + nbuf - 1)

    b = i // nh
    h0 = (i % nh) * hb
    slot = i % nbuf
    pltpu.make_async_copy(
        x_hbm.at[b, :, pl.ds(h0, hb), :],
        xbuf.at[slot],
        sems.at[slot],
    ).wait()

    xb = xbuf[slot]
    proto = proto_ref[...]
    p_sq = jnp.sum(proto * proto, axis=1)[:, None, None]
    dots = jax.lax.dot_general(
        proto, xb, (((1,), (0,)), ((), ())),
        preferred_element_type=jnp.float32)
    sq = p_sq - 2.0 * dots
    kp, _, w = sq.shape
    sqm = jnp.max(sq.reshape(num_classes, kp // num_classes, hb, w), axis=1)
    x_sq = jnp.sum(xb * xb, axis=0, keepdims=True)
    out_ref[b, :, pl.ds(h0, hb), :] = jnp.sqrt(jnp.maximum(sqm + x_sq, 0.0))


def kernel(x, gt, prototype):
    del gt
    B, C, H, W = x.shape
    K, P, _ = prototype.shape
    KP = K * P

    proto = prototype.reshape(KP, C)
    nchunk = B * (H // _HB)

    return pl.pallas_call(
        functools.partial(_ecc_manual_kernel, num_classes=K, hb=_HB,
                          nbuf=_NBUF),
        grid=(nchunk,),
        in_specs=[
            pl.BlockSpec(memory_space=pltpu.HBM),
            pl.BlockSpec((KP, C), lambda i: (0, 0)),
        ],
        out_specs=pl.BlockSpec((B, K, H, W), lambda i: (0, 0, 0, 0)),
        out_shape=jax.ShapeDtypeStruct((B, K, H, W), jnp.float32),
        scratch_shapes=[
            pltpu.VMEM((_NBUF, C, _HB, W), jnp.float32),
            pltpu.SemaphoreType.DMA((_NBUF,)),
        ],
    )(x, proto)


# speedup vs baseline: 1.0120x; 1.0120x over previous
import functools

import jax
import jax.numpy as jnp
from jax.experimental import pallas as pl
from jax.experimental.pallas import tpu as pltpu

_HB = 8
_NBUF = 12


def _copy(x_hbm, xbuf, sems, j, slot, hb, nh):
    jb = j // nh
    jh = j % nh
    return pltpu.make_async_copy(
        x_hbm.at[jb, :, pl.ds(jh * hb, hb), :],
        xbuf.at[slot],
        sems.at[slot],
    )


def _ecc_manual_kernel(x_hbm, proto_ref, out_ref, xbuf, sems, *,
                       num_classes, hb, nbuf):
    s = pl.program_id(0)
    nrounds = pl.num_programs(0)
    nh = out_ref.shape[2] // hb

    @pl.when(s == 0)
    def _():
        for slot in range(nbuf):
            _copy(x_hbm, xbuf, sems, slot, slot, hb, nh).start()

    proto = proto_ref[...]
    p_sq = jnp.sum(proto * proto, axis=1)[:, None, None]

    for slot in range(nbuf):
        j = s * nbuf + slot
        _copy(x_hbm, xbuf, sems, j, slot, hb, nh).wait()
        xb = xbuf[slot]

        @pl.when(s + 1 < nrounds)
        def _():
            _copy(x_hbm, xbuf, sems, j + nbuf, slot, hb, nh).start()

        dots = jax.lax.dot_general(
            proto, xb, (((1,), (0,)), ((), ())),
            preferred_element_type=jnp.float32)
        sq = p_sq - 2.0 * dots
        kp, _, w = sq.shape
        sqm = jnp.max(sq.reshape(num_classes, kp // num_classes, hb, w),
                      axis=1)
        x_sq = jnp.sum(xb * xb, axis=0, keepdims=True)
        b = j // nh
        h0 = (j % nh) * hb
        out_ref[b, :, pl.ds(h0, hb), :] = jnp.sqrt(
            jnp.maximum(sqm + x_sq, 0.0))


def kernel(x, gt, prototype):
    del gt
    B, C, H, W = x.shape
    K, P, _ = prototype.shape
    KP = K * P

    proto = prototype.reshape(KP, C)
    nchunk = B * (H // _HB)

    return pl.pallas_call(
        functools.partial(_ecc_manual_kernel, num_classes=K, hb=_HB,
                          nbuf=_NBUF),
        grid=(nchunk // _NBUF,),
        in_specs=[
            pl.BlockSpec(memory_space=pltpu.HBM),
            pl.BlockSpec((KP, C), lambda i: (0, 0)),
        ],
        out_specs=pl.BlockSpec((B, K, H, W), lambda i: (0, 0, 0, 0)),
        out_shape=jax.ShapeDtypeStruct((B, K, H, W), jnp.float32),
        scratch_shapes=[
            pltpu.VMEM((_NBUF, C, _HB, W), jnp.float32),
            pltpu.SemaphoreType.DMA((_NBUF,)),
        ],
    )(x, proto)


# speedup vs baseline: 1.4527x; 1.4355x over previous
import functools

import jax
import jax.numpy as jnp
from jax.experimental import pallas as pl
from jax.experimental.pallas import tpu as pltpu


def _ecc_block_kernel(x_ref, proto_ref, out_ref, *, num_classes):
    xb = x_ref[0]
    proto = proto_ref[...]
    p_sq = jnp.sum(proto * proto, axis=1, keepdims=True)
    dots = jax.lax.dot_general(
        proto, xb, (((1,), (0,)), ((), ())),
        preferred_element_type=jnp.float32)
    sq = p_sq - 2.0 * dots
    kp, t = sq.shape
    sqm = jnp.max(sq.reshape(num_classes, kp // num_classes, t), axis=1)
    x_sq = jnp.sum(xb * xb, axis=0, keepdims=True)
    out_ref[0] = jnp.sqrt(jnp.maximum(sqm + x_sq, 0.0))


def kernel(x, gt, prototype):
    del gt
    B, C, H, W = x.shape
    K, P, _ = prototype.shape
    KP = K * P
    HW = H * W
    T = 2304

    xr = x.reshape(B, C, HW)
    proto = prototype.reshape(KP, C)

    out = pl.pallas_call(
        functools.partial(_ecc_block_kernel, num_classes=K),
        grid=(B, HW // T),
        in_specs=[
            pl.BlockSpec((1, C, T), lambda b, t: (b, 0, t)),
            pl.BlockSpec((KP, C), lambda b, t: (0, 0)),
        ],
        out_specs=pl.BlockSpec((1, K, T), lambda b, t: (b, 0, t)),
        out_shape=jax.ShapeDtypeStruct((B, K, HW), jnp.float32),
        compiler_params=pltpu.CompilerParams(
            allow_input_fusion=(True, False),
        ),
    )(xr, proto)
    return out.reshape(B, K, H, W)
